# baseline (speedup 1.0000x reference)
import functools

import jax
import jax.numpy as jnp
from jax import lax
from jax.experimental import pallas as pl
from jax.experimental.pallas import tpu as pltpu
from jax.experimental.pallas import tpu_sc as plsc

D, F = 768, 512
NE = 8
EPS = 1e-06
SCALE = 2.5
TB = 256
SB = 256
NSLOT_BLK = 24
RSLOTS = NSLOT_BLK * SB
LANES = 128
NW = 32
TPW = 64
CHK = 16


def _k1_body(h_ref, nw_ref, wr_ref, bias_ref, wsg_ref, wsu_ref, wsd_ref,
             normed_ref, hs_ref, w_ref, pos_ref, be_ref, cnt_ref):
    i = pl.program_id(0)

    @pl.when(i == 0)
    def _init():
        cnt_ref[...] = jnp.zeros_like(cnt_ref)

    @pl.when(i < 8)
    def _token_block():
        x = h_ref[...]
        var = jnp.mean(x * x, axis=1, keepdims=True)
        normed = x * lax.rsqrt(var + EPS) * nw_ref[...]
        normed_ref[...] = normed

        sg = jnp.dot(normed, wsg_ref[...], preferred_element_type=jnp.float32)
        su = jnp.dot(normed, wsu_ref[...], preferred_element_type=jnp.float32)
        hs_ref[...] = x + jnp.dot(jax.nn.silu(sg) * su, wsd_ref[...],
                                  preferred_element_type=jnp.float32)

        logits = jnp.dot(normed, wr_ref[...], preferred_element_type=jnp.float32)
        scores = jax.nn.sigmoid(logits)
        lane = lax.broadcasted_iota(jnp.int32, (TB, LANES), 1)
        valid = lane < NE
        biased = jnp.where(valid, scores + bias_ref[...], -1e30)

        m1 = jnp.max(biased, axis=1, keepdims=True)
        idx1 = jnp.min(jnp.where(biased == m1, lane, LANES), axis=1,
                       keepdims=True)
        oh1 = lane == idx1
        biased2 = jnp.where(oh1, -1e30, biased)
        m2 = jnp.max(biased2, axis=1, keepdims=True)
        idx2 = jnp.min(jnp.where(biased2 == m2, lane, LANES), axis=1,
                       keepdims=True)
        oh2 = lane == idx2

        w1 = jnp.sum(jnp.where(oh1, scores, 0.0), axis=1, keepdims=True)
        w2 = jnp.sum(jnp.where(oh2, scores, 0.0), axis=1, keepdims=True)
        denom = w1 + w2 + 1e-9
        w1 = w1 / denom * SCALE
        w2 = w2 / denom * SCALE
        w_ref[...] = jnp.where(lane < 16, w1,
                               jnp.where(lane < 32, w2, 0.0))

        a = oh1.astype(jnp.float32) + oh2.astype(jnp.float32)
        r_io = lax.broadcasted_iota(jnp.int32, (TB, TB), 0)
        c_io = lax.broadcasted_iota(jnp.int32, (TB, TB), 1)
        ltri = (c_io < r_io).astype(jnp.float32)
        excl = jnp.dot(ltri, a, preferred_element_type=jnp.float32)
        tot = excl + cnt_ref[0:1, :]
        rank1 = jnp.sum(jnp.where(oh1, tot, 0.0), axis=1, keepdims=True)
        rank2 = jnp.sum(jnp.where(oh2, tot, 0.0), axis=1, keepdims=True)
        cnt_ref[0:1, :] = cnt_ref[0:1, :] + jnp.sum(a, axis=0, keepdims=True)

        vals = (jnp.where(lane == 0, rank1.astype(jnp.int32), 0) +
                jnp.where(lane == 1, rank2.astype(jnp.int32), 0) +
                jnp.where(lane == 2, idx1, 0) +
                jnp.where(lane == 3, idx2, 0))
        pos_ref[pl.ds(i * TB, TB), :] = vals

    @pl.when(i == 8)
    def _finalize():
        c = cnt_ref[0:1, :]
        nb = jnp.floor((c + (SB - 1.0)) / SB)
        r_io = lax.broadcasted_iota(jnp.int32, (LANES, LANES), 0)
        c_io = lax.broadcasted_iota(jnp.int32, (LANES, LANES), 1)
        utri = (r_io <= c_io).astype(jnp.float32)
        bc = jnp.dot(nb, utri, preferred_element_type=jnp.float32)
        base = (bc - nb) * SB

        px = pos_ref[...]
        t = px.shape[0]
        rank1 = px[:, 0:1].astype(jnp.float32)
        rank2 = px[:, 1:2].astype(jnp.float32)
        idx1 = px[:, 2:3]
        idx2 = px[:, 3:4]
        lane_t = lax.broadcasted_iota(jnp.int32, (t, LANES), 1)
        b1 = jnp.sum(jnp.where(lane_t == idx1, base, 0.0), axis=1,
                     keepdims=True)
        b2 = jnp.sum(jnp.where(lane_t == idx2, base, 0.0), axis=1,
                     keepdims=True)
        pos1 = (rank1 + b1).astype(jnp.int32)
        pos2 = (rank2 + b2).astype(jnp.int32)
        pos_ref[...] = (jnp.where(lane_t == 0, pos1, 0) +
                        jnp.where(lane_t == 1, pos2, 0))

        lane_r = lax.broadcasted_iota(jnp.int32, (1, LANES), 1).astype(jnp.float32)
        bev = jnp.zeros((1, LANES), jnp.float32)
        for e in range(NE):
            bc_e = jnp.sum(jnp.where(lane_r == e, bc, 0.0), axis=1,
                           keepdims=True)
            bev = bev + (lane_r >= bc_e).astype(jnp.float32)
        be_ref[...] = jnp.minimum(bev, 7.0).astype(jnp.int32)


def _stage1(h, norm_w, Wr, expert_bias, Wsg, Wsu, Wsd):
    t, d = h.shape
    wr_pad = jnp.zeros((d, LANES), jnp.float32).at[:, :NE].set(Wr)
    bias_pad = jnp.zeros((1, LANES), jnp.float32).at[0, :NE].set(expert_bias)
    nblk = t // TB
    blk_map = lambda i: (jnp.minimum(i, nblk - 1), 0)
    return pl.pallas_call(
        _k1_body,
        grid=(nblk + 1,),
        in_specs=[
            pl.BlockSpec((TB, d), blk_map),
            pl.BlockSpec((1, d), lambda i: (0, 0)),
            pl.BlockSpec((d, LANES), lambda i: (0, 0)),
            pl.BlockSpec((1, LANES), lambda i: (0, 0)),
            pl.BlockSpec((d, F), lambda i: (0, 0)),
            pl.BlockSpec((d, F), lambda i: (0, 0)),
            pl.BlockSpec((F, d), lambda i: (0, 0)),
        ],
        out_specs=[
            pl.BlockSpec((TB, d), blk_map),
            pl.BlockSpec((TB, d), blk_map),
            pl.BlockSpec((TB, LANES), blk_map),
            pl.BlockSpec((t, LANES), lambda i: (0, 0)),
            pl.BlockSpec((1, LANES), lambda i: (0, 0)),
        ],
        out_shape=[
            jax.ShapeDtypeStruct((t, d), jnp.float32),
            jax.ShapeDtypeStruct((t, d), jnp.float32),
            jax.ShapeDtypeStruct((t, LANES), jnp.float32),
            jax.ShapeDtypeStruct((t, LANES), jnp.int32),
            jax.ShapeDtypeStruct((1, LANES), jnp.int32),
        ],
        scratch_shapes=[pltpu.VMEM((8, LANES), jnp.float32)],
        compiler_params=pltpu.CompilerParams(
            dimension_semantics=("arbitrary",),
        ),
    )(h, norm_w.reshape(1, d), wr_pad, bias_pad, Wsg, Wsu, Wsd)


def _k3_body(be_ref, x_ref, wg_ref, wu_ref, wd_ref, y_ref):
    x = x_ref[...]
    g = jnp.dot(x, wg_ref[0], preferred_element_type=jnp.float32)
    u = jnp.dot(x, wu_ref[0], preferred_element_type=jnp.float32)
    y_ref[...] = jnp.dot(jax.nn.silu(g) * u, wd_ref[0],
                         preferred_element_type=jnp.float32)


def _stage3(be32, buf, Wg, Wu, Wd):
    return pl.pallas_call(
        _k3_body,
        grid_spec=pltpu.PrefetchScalarGridSpec(
            num_scalar_prefetch=1,
            grid=(NSLOT_BLK,),
            in_specs=[
                pl.BlockSpec((SB, D), lambda i, be: (i, 0)),
                pl.BlockSpec((1, D, F), lambda i, be: (be[i], 0, 0)),
                pl.BlockSpec((1, D, F), lambda i, be: (be[i], 0, 0)),
                pl.BlockSpec((1, F, D), lambda i, be: (be[i], 0, 0)),
            ],
            out_specs=pl.BlockSpec((SB, D), lambda i, be: (i, 0)),
        ),
        out_shape=jax.ShapeDtypeStruct((RSLOTS, D), jnp.float32),
        compiler_params=pltpu.CompilerParams(
            dimension_semantics=("arbitrary",),
        ),
    )(be32, buf, Wg, Wu, Wd)


def _dispatch_sc(normed, pos_t):
    mesh = plsc.VectorSubcoreMesh(core_axis_name="c", subcore_axis_name="s")
    info = plsc.get_sparse_core_info()
    nc = info.num_cores

    @functools.partial(
        pl.kernel, mesh=mesh,
        out_type=jax.ShapeDtypeStruct((RSLOTS, D), jnp.float32),
        scratch_types=[
            pltpu.VMEM((TPW, D), jnp.float32),
            pltpu.VMEM((TPW,), jnp.int32),
            pltpu.VMEM((TPW,), jnp.int32),
            pltpu.SemaphoreType.DMA,
        ],
    )
    def k2(normed_hbm, pos_hbm, buf_hbm, xbuf, p1, p2, sem):
        wid = lax.axis_index("s") * nc + lax.axis_index("c")
        base = wid * TPW
        pltpu.sync_copy(normed_hbm.at[pl.ds(base, TPW)], xbuf)
        pltpu.sync_copy(pos_hbm.at[0, pl.ds(base, TPW)], p1)
        pltpu.sync_copy(pos_hbm.at[1, pl.ds(base, TPW)], p2)
        pltpu.async_copy(xbuf, buf_hbm.at[p1], sem).wait()
        pltpu.async_copy(xbuf, buf_hbm.at[p2], sem).wait()

    return k2(normed, pos_t)


def _combine_sc(hs, y, pos_t, w1e, w2e):
    mesh = plsc.VectorSubcoreMesh(core_axis_name="c", subcore_axis_name="s")
    info = plsc.get_sparse_core_info()
    nc = info.num_cores
    t, d = hs.shape
    nvec = d // 16

    @functools.partial(
        pl.kernel, mesh=mesh,
        out_type=jax.ShapeDtypeStruct((t, d), jnp.float32),
        scratch_types=[
            pltpu.VMEM((TPW,), jnp.int32),
            pltpu.VMEM((TPW,), jnp.int32),
            pltpu.VMEM((TPW * 16,), jnp.float32),
            pltpu.VMEM((TPW * 16,), jnp.float32),
            pltpu.VMEM((CHK, D), jnp.float32),
            pltpu.VMEM((CHK, D), jnp.float32),
            pltpu.VMEM((CHK, D), jnp.float32),
            pltpu.VMEM((CHK, D), jnp.float32),
            pltpu.SemaphoreType.DMA,
            pltpu.SemaphoreType.DMA,
            pltpu.SemaphoreType.DMA,
        ],
    )
    def k4(hs_hbm, y_hbm, pos_hbm, w1_hbm, w2_hbm, out_hbm,
           p1, p2, wv1, wv2, y1b, y2b, hsb, ob, s1, s2, s3):
        wid = lax.axis_index("s") * nc + lax.axis_index("c")
        base = wid * TPW
        pltpu.sync_copy(pos_hbm.at[0, pl.ds(base, TPW)], p1)
        pltpu.sync_copy(pos_hbm.at[1, pl.ds(base, TPW)], p2)
        pltpu.sync_copy(w1_hbm.at[pl.ds(base * 16, TPW * 16)], wv1)
        pltpu.sync_copy(w2_hbm.at[pl.ds(base * 16, TPW * 16)], wv2)

        for c in range(TPW // CHK):
            idx1 = p1[pl.ds(c * CHK, CHK)]
            idx2 = p2[pl.ds(c * CHK, CHK)]
            cp1 = pltpu.async_copy(y_hbm.at[idx1], y1b, s1)
            cp2 = pltpu.async_copy(y_hbm.at[idx2], y2b, s2)
            cp3 = pltpu.async_copy(hs_hbm.at[pl.ds(base + c * CHK, CHK)],
                                   hsb, s3)
            cp1.wait()
            cp2.wait()
            cp3.wait()
            wb1 = []
            wb2 = []
            for t2 in range(CHK):
                tok = c * CHK + t2
                wb1.append(wv1[pl.ds(tok * 16, 16)])
                wb2.append(wv2[pl.ds(tok * 16, 16)])

            def body(j, carry):
                sl = pl.ds(j * 16, 16)
                for t2 in range(CHK):
                    ob[t2, sl] = (hsb[t2, sl] + wb1[t2] * y1b[t2, sl] +
                                  wb2[t2] * y2b[t2, sl])
                return carry

            lax.fori_loop(0, nvec, body, 0)
            pltpu.sync_copy(ob, out_hbm.at[pl.ds(base + c * CHK, CHK)])

    return k4(hs, y, pos_t, w1e, w2e)


@jax.jit
def kernel(hidden_states, norm_w, Wr, expert_bias, Wg, Wu, Wd, Wsg, Wsu, Wsd):
    b, s, d = hidden_states.shape
    h = hidden_states.reshape(b * s, d)

    normed, hs, w_l, pos_l, be_l = _stage1(
        h, norm_w, Wr, expert_bias, Wsg, Wsu, Wsd)

    pos_t = pos_l[:, :2].T
    w1e = w_l[:, :16].reshape(b * s * 16)
    w2e = w_l[:, 16:32].reshape(b * s * 16)
    be32 = be_l[0, :NSLOT_BLK]

    buf = _dispatch_sc(normed, pos_t)
    y = _stage3(be32, buf, Wg, Wu, Wd)
    out = _combine_sc(hs, y, pos_t, w1e, w2e)
    return out.reshape(b, s, d)


# speedup vs baseline: 1.0356x; 1.0356x over previous
import functools

import jax
import jax.numpy as jnp
from jax import lax
from jax.experimental import pallas as pl
from jax.experimental.pallas import tpu as pltpu
from jax.experimental.pallas import tpu_sc as plsc

D, F = 768, 512
NE = 8
EPS = 1e-06
SCALE = 2.5
TB = 256
SB = 256
NSLOT_BLK = 24
RSLOTS = NSLOT_BLK * SB
LANES = 128
NW = 32
TPW = 64
CHK = 16


def _k1_body(h_ref, nw_ref, wr_ref, bias_ref, wsg_ref, wsu_ref, wsd_ref,
             normed_ref, hs_ref, w_ref, pos_ref, be_ref, cnt_ref):
    i = pl.program_id(0)

    @pl.when(i == 0)
    def _init():
        cnt_ref[...] = jnp.zeros_like(cnt_ref)

    @pl.when(i < 8)
    def _token_block():
        x = h_ref[...]
        var = jnp.mean(x * x, axis=1, keepdims=True)
        normed = x * lax.rsqrt(var + EPS) * nw_ref[...]
        normed_ref[...] = normed

        sg = jnp.dot(normed, wsg_ref[...], preferred_element_type=jnp.float32)
        su = jnp.dot(normed, wsu_ref[...], preferred_element_type=jnp.float32)
        hs_ref[...] = x + jnp.dot(jax.nn.silu(sg) * su, wsd_ref[...],
                                  preferred_element_type=jnp.float32)

        logits = jnp.dot(normed, wr_ref[...], preferred_element_type=jnp.float32)
        scores = jax.nn.sigmoid(logits)
        lane = lax.broadcasted_iota(jnp.int32, (TB, LANES), 1)
        valid = lane < NE
        biased = jnp.where(valid, scores + bias_ref[...], -1e30)

        m1 = jnp.max(biased, axis=1, keepdims=True)
        idx1 = jnp.min(jnp.where(biased == m1, lane, LANES), axis=1,
                       keepdims=True)
        oh1 = lane == idx1
        biased2 = jnp.where(oh1, -1e30, biased)
        m2 = jnp.max(biased2, axis=1, keepdims=True)
        idx2 = jnp.min(jnp.where(biased2 == m2, lane, LANES), axis=1,
                       keepdims=True)
        oh2 = lane == idx2

        w1 = jnp.sum(jnp.where(oh1, scores, 0.0), axis=1, keepdims=True)
        w2 = jnp.sum(jnp.where(oh2, scores, 0.0), axis=1, keepdims=True)
        denom = w1 + w2 + 1e-9
        w1 = w1 / denom * SCALE
        w2 = w2 / denom * SCALE
        w_ref[...] = jnp.where(lane < 16, w1,
                               jnp.where(lane < 32, w2, 0.0))

        a = oh1.astype(jnp.float32) + oh2.astype(jnp.float32)
        r_io = lax.broadcasted_iota(jnp.int32, (TB, TB), 0)
        c_io = lax.broadcasted_iota(jnp.int32, (TB, TB), 1)
        ltri = (c_io < r_io).astype(jnp.float32)
        excl = jnp.dot(ltri, a, preferred_element_type=jnp.float32)
        tot = excl + cnt_ref[0:1, :]
        rank1 = jnp.sum(jnp.where(oh1, tot, 0.0), axis=1, keepdims=True)
        rank2 = jnp.sum(jnp.where(oh2, tot, 0.0), axis=1, keepdims=True)
        cnt_ref[0:1, :] = cnt_ref[0:1, :] + jnp.sum(a, axis=0, keepdims=True)

        vals = (jnp.where(lane == 0, rank1.astype(jnp.int32), 0) +
                jnp.where(lane == 1, rank2.astype(jnp.int32), 0) +
                jnp.where(lane == 2, idx1, 0) +
                jnp.where(lane == 3, idx2, 0))
        pos_ref[pl.ds(i * TB, TB), :] = vals

    @pl.when(i == 8)
    def _finalize():
        c = cnt_ref[0:1, :]
        nb = jnp.floor((c + (SB - 1.0)) / SB)
        r_io = lax.broadcasted_iota(jnp.int32, (LANES, LANES), 0)
        c_io = lax.broadcasted_iota(jnp.int32, (LANES, LANES), 1)
        utri = (r_io <= c_io).astype(jnp.float32)
        bc = jnp.dot(nb, utri, preferred_element_type=jnp.float32)
        base = (bc - nb) * SB

        px = pos_ref[...]
        t = px.shape[0]
        rank1 = px[:, 0:1].astype(jnp.float32)
        rank2 = px[:, 1:2].astype(jnp.float32)
        idx1 = px[:, 2:3]
        idx2 = px[:, 3:4]
        lane_t = lax.broadcasted_iota(jnp.int32, (t, LANES), 1)
        b1 = jnp.sum(jnp.where(lane_t == idx1, base, 0.0), axis=1,
                     keepdims=True)
        b2 = jnp.sum(jnp.where(lane_t == idx2, base, 0.0), axis=1,
                     keepdims=True)
        pos1 = (rank1 + b1).astype(jnp.int32)
        pos2 = (rank2 + b2).astype(jnp.int32)
        pos_ref[...] = (jnp.where(lane_t == 0, pos1, 0) +
                        jnp.where(lane_t == 1, pos2, 0))

        lane_r = lax.broadcasted_iota(jnp.int32, (1, LANES), 1).astype(jnp.float32)
        bev = jnp.zeros((1, LANES), jnp.float32)
        for e in range(NE):
            bc_e = jnp.sum(jnp.where(lane_r == e, bc, 0.0), axis=1,
                           keepdims=True)
            bev = bev + (lane_r >= bc_e).astype(jnp.float32)
        be_ref[...] = jnp.minimum(bev, 7.0).astype(jnp.int32)


def _stage1(h, norm_w, Wr, expert_bias, Wsg, Wsu, Wsd):
    t, d = h.shape
    wr_pad = jnp.zeros((d, LANES), jnp.float32).at[:, :NE].set(Wr)
    bias_pad = jnp.zeros((1, LANES), jnp.float32).at[0, :NE].set(expert_bias)
    nblk = t // TB
    blk_map = lambda i: (jnp.minimum(i, nblk - 1), 0)
    return pl.pallas_call(
        _k1_body,
        grid=(nblk + 1,),
        in_specs=[
            pl.BlockSpec((TB, d), blk_map),
            pl.BlockSpec((1, d), lambda i: (0, 0)),
            pl.BlockSpec((d, LANES), lambda i: (0, 0)),
            pl.BlockSpec((1, LANES), lambda i: (0, 0)),
            pl.BlockSpec((d, F), lambda i: (0, 0)),
            pl.BlockSpec((d, F), lambda i: (0, 0)),
            pl.BlockSpec((F, d), lambda i: (0, 0)),
        ],
        out_specs=[
            pl.BlockSpec((TB, d), blk_map),
            pl.BlockSpec((TB, d), blk_map),
            pl.BlockSpec((TB, LANES), blk_map),
            pl.BlockSpec((t, LANES), lambda i: (0, 0)),
            pl.BlockSpec((1, LANES), lambda i: (0, 0)),
        ],
        out_shape=[
            jax.ShapeDtypeStruct((t, d), jnp.float32),
            jax.ShapeDtypeStruct((t, d), jnp.float32),
            jax.ShapeDtypeStruct((t, LANES), jnp.float32),
            jax.ShapeDtypeStruct((t, LANES), jnp.int32),
            jax.ShapeDtypeStruct((1, LANES), jnp.int32),
        ],
        scratch_shapes=[pltpu.VMEM((8, LANES), jnp.float32)],
        compiler_params=pltpu.CompilerParams(
            dimension_semantics=("arbitrary",),
        ),
    )(h, norm_w.reshape(1, d), wr_pad, bias_pad, Wsg, Wsu, Wsd)


def _k3_body(be_ref, x_ref, wg_ref, wu_ref, wd_ref, y_ref):
    e = be_ref[pl.program_id(0)]
    x = x_ref[...]
    g = jnp.dot(x, wg_ref[e], preferred_element_type=jnp.float32)
    u = jnp.dot(x, wu_ref[e], preferred_element_type=jnp.float32)
    y_ref[...] = jnp.dot(jax.nn.silu(g) * u, wd_ref[e],
                         preferred_element_type=jnp.float32)


def _stage3(be32, buf, Wg, Wu, Wd):
    return pl.pallas_call(
        _k3_body,
        grid_spec=pltpu.PrefetchScalarGridSpec(
            num_scalar_prefetch=1,
            grid=(NSLOT_BLK,),
            in_specs=[
                pl.BlockSpec((SB, D), lambda i, be: (i, 0)),
                pl.BlockSpec((NE, D, F), lambda i, be: (0, 0, 0)),
                pl.BlockSpec((NE, D, F), lambda i, be: (0, 0, 0)),
                pl.BlockSpec((NE, F, D), lambda i, be: (0, 0, 0)),
            ],
            out_specs=pl.BlockSpec((SB, D), lambda i, be: (i, 0)),
        ),
        out_shape=jax.ShapeDtypeStruct((RSLOTS, D), jnp.float32),
        compiler_params=pltpu.CompilerParams(
            dimension_semantics=("arbitrary",),
        ),
    )(be32, buf, Wg, Wu, Wd)


def _dispatch_sc(normed, pos_t):
    mesh = plsc.VectorSubcoreMesh(core_axis_name="c", subcore_axis_name="s")
    info = plsc.get_sparse_core_info()
    nc = info.num_cores

    @functools.partial(
        pl.kernel, mesh=mesh,
        out_type=jax.ShapeDtypeStruct((RSLOTS, D), jnp.float32),
        scratch_types=[
            pltpu.VMEM((TPW, D), jnp.float32),
            pltpu.VMEM((TPW,), jnp.int32),
            pltpu.VMEM((TPW,), jnp.int32),
            pltpu.SemaphoreType.DMA,
        ],
    )
    def k2(normed_hbm, pos_hbm, buf_hbm, xbuf, p1, p2, sem):
        wid = lax.axis_index("s") * nc + lax.axis_index("c")
        base = wid * TPW
        pltpu.sync_copy(normed_hbm.at[pl.ds(base, TPW)], xbuf)
        pltpu.sync_copy(pos_hbm.at[0, pl.ds(base, TPW)], p1)
        pltpu.sync_copy(pos_hbm.at[1, pl.ds(base, TPW)], p2)
        pltpu.async_copy(xbuf, buf_hbm.at[p1], sem).wait()
        pltpu.async_copy(xbuf, buf_hbm.at[p2], sem).wait()

    return k2(normed, pos_t)


def _combine_sc(hs, y, pos_t, w1e, w2e):
    mesh = plsc.VectorSubcoreMesh(core_axis_name="c", subcore_axis_name="s")
    info = plsc.get_sparse_core_info()
    nc = info.num_cores
    t, d = hs.shape
    nvec = d // 16

    @functools.partial(
        pl.kernel, mesh=mesh,
        out_type=jax.ShapeDtypeStruct((t, d), jnp.float32),
        scratch_types=[
            pltpu.VMEM((TPW,), jnp.int32),
            pltpu.VMEM((TPW,), jnp.int32),
            pltpu.VMEM((TPW * 16,), jnp.float32),
            pltpu.VMEM((TPW * 16,), jnp.float32),
            pltpu.VMEM((2, CHK, D), jnp.float32),
            pltpu.VMEM((2, CHK, D), jnp.float32),
            pltpu.VMEM((2, CHK, D), jnp.float32),
            pltpu.VMEM((CHK, D), jnp.float32),
            pltpu.SemaphoreType.DMA,
            pltpu.SemaphoreType.DMA,
            pltpu.SemaphoreType.DMA,
            pltpu.SemaphoreType.DMA,
            pltpu.SemaphoreType.DMA,
            pltpu.SemaphoreType.DMA,
        ],
    )
    def k4(hs_hbm, y_hbm, pos_hbm, w1_hbm, w2_hbm, out_hbm,
           p1, p2, wv1, wv2, y1b, y2b, hsb, ob, s1a, s2a, s3a, s1b, s2b, s3b):
        wid = lax.axis_index("s") * nc + lax.axis_index("c")
        base = wid * TPW
        pltpu.sync_copy(pos_hbm.at[0, pl.ds(base, TPW)], p1)
        pltpu.sync_copy(pos_hbm.at[1, pl.ds(base, TPW)], p2)
        pltpu.sync_copy(w1_hbm.at[pl.ds(base * 16, TPW * 16)], wv1)
        pltpu.sync_copy(w2_hbm.at[pl.ds(base * 16, TPW * 16)], wv2)
        sems = [(s1a, s2a, s3a), (s1b, s2b, s3b)]
        nchunk = TPW // CHK

        def start(c):
            sl = c % 2
            s1, s2, s3 = sems[sl]
            cps = (pltpu.async_copy(y_hbm.at[p1[pl.ds(c * CHK, CHK)]],
                                    y1b.at[sl], s1),
                   pltpu.async_copy(y_hbm.at[p2[pl.ds(c * CHK, CHK)]],
                                    y2b.at[sl], s2),
                   pltpu.async_copy(hs_hbm.at[pl.ds(base + c * CHK, CHK)],
                                    hsb.at[sl], s3))
            return cps

        inflight = start(0)
        for c in range(nchunk):
            for cp in inflight:
                cp.wait()
            if c + 1 < nchunk:
                nxt = start(c + 1)
            sl = c % 2
            wb1 = []
            wb2 = []
            for t2 in range(CHK):
                tok = c * CHK + t2
                wb1.append(wv1[pl.ds(tok * 16, 16)])
                wb2.append(wv2[pl.ds(tok * 16, 16)])

            def body(j, carry):
                s = pl.ds(j * 16, 16)
                for t2 in range(CHK):
                    ob[t2, s] = (hsb[sl, t2, s] + wb1[t2] * y1b[sl, t2, s] +
                                 wb2[t2] * y2b[sl, t2, s])
                return carry

            lax.fori_loop(0, nvec, body, 0)
            pltpu.sync_copy(ob, out_hbm.at[pl.ds(base + c * CHK, CHK)])
            if c + 1 < nchunk:
                inflight = nxt

    return k4(hs, y, pos_t, w1e, w2e)


@jax.jit
def kernel(hidden_states, norm_w, Wr, expert_bias, Wg, Wu, Wd, Wsg, Wsu, Wsd):
    b, s, d = hidden_states.shape
    h = hidden_states.reshape(b * s, d)

    normed, hs, w_l, pos_l, be_l = _stage1(
        h, norm_w, Wr, expert_bias, Wsg, Wsu, Wsd)

    pos_t = pos_l[:, :2].T
    w1e = w_l[:, :16].reshape(b * s * 16)
    w2e = w_l[:, 16:32].reshape(b * s * 16)
    be32 = be_l[0, :NSLOT_BLK]

    buf = _dispatch_sc(normed, pos_t)
    y = _stage3(be32, buf, Wg, Wu, Wd)
    out = _combine_sc(hs, y, pos_t, w1e, w2e)
    return out.reshape(b, s, d)
